# baseline (speedup 1.0000x reference)
import functools

import jax
import jax.numpy as jnp
from jax import lax
from jax.experimental import pallas as pl
from jax.experimental.pallas import tpu as pltpu
from jax.experimental.pallas import tpu_sc as plsc

_BM = 256
_BN = 2048

_NC = 2
_NS = 16
_NW = _NC * _NS


def _dist_argmin_body(cbp_ref, zp_ref, zsq_ref, csqw_ref,
                      idx_ref, loss_ref, runmin_ref, runidx_ref, truemin_ref):
    i = pl.program_id(0)
    j = pl.program_id(1)
    mt = lax.dot_general(cbp_ref[...], zp_ref[...], (((1,), (0,)), ((), ())),
                         preferred_element_type=jnp.float32)
    d = (zsq_ref[...] + csqw_ref[...]) - 2.0 * mt
    bmin = jnp.min(d, axis=0, keepdims=True)
    row = lax.broadcasted_iota(jnp.int32, d.shape, 0)
    bidx = jnp.min(jnp.where(d == bmin, row, jnp.int32(2**30)),
                   axis=0, keepdims=True) + j * _BN
    bmin_bf = bmin.astype(jnp.bfloat16).astype(jnp.float32)

    @pl.when(j == 0)
    def _init():
        runmin_ref[...] = bmin_bf
        runidx_ref[...] = bidx
        truemin_ref[...] = bmin

    @pl.when(j > 0)
    def _update():
        better = bmin < runmin_ref[...]
        runidx_ref[...] = jnp.where(better, bidx, runidx_ref[...])
        runmin_ref[...] = jnp.where(better, bmin_bf, runmin_ref[...])
        truemin_ref[...] = jnp.minimum(truemin_ref[...], bmin)

    @pl.when(j == pl.num_programs(1) - 1)
    def _emit():
        idx_ref[...] = runidx_ref[...].reshape(-1)
        s = jnp.sum(truemin_ref[...])

        @pl.when(i == 0)
        def _first():
            loss_ref[0, 0] = s

        @pl.when(i > 0)
        def _acc():
            loss_ref[0, 0] = loss_ref[0, 0] + s


def _dist_argmin(cb_packed, z_packed, zsq_row, csq_wide, interpret=False):
    nk, k2 = cb_packed.shape
    n = z_packed.shape[1]
    grid = (n // _BM, nk // _BN)
    return pl.pallas_call(
        _dist_argmin_body,
        grid=grid,
        in_specs=[
            pl.BlockSpec((_BN, k2), lambda i, j: (j, 0)),
            pl.BlockSpec((k2, _BM), lambda i, j: (0, i)),
            pl.BlockSpec((1, _BM), lambda i, j: (0, i)),
            pl.BlockSpec((_BN, _BM), lambda i, j: (j, 0)),
        ],
        out_specs=[
            pl.BlockSpec((_BM,), lambda i, j: (i,)),
            pl.BlockSpec(memory_space=pltpu.SMEM),
        ],
        out_shape=[
            jax.ShapeDtypeStruct((n,), jnp.int32),
            jax.ShapeDtypeStruct((1, 1), jnp.float32),
        ],
        scratch_shapes=[
            pltpu.VMEM((1, _BM), jnp.float32),
            pltpu.VMEM((1, _BM), jnp.int32),
            pltpu.VMEM((1, _BM), jnp.float32),
        ],
        compiler_params=pltpu.CompilerParams(
            dimension_semantics=("arbitrary", "arbitrary")),
        interpret=interpret,
    )(cb_packed, z_packed, zsq_row, csq_wide)


def _sc_gather(codebook_padded, idx):
    bn = idx.shape[0]
    d = codebook_padded.shape[1]
    b_per_w = bn // _NW
    nchunk = b_per_w // 128
    idx_r = idx.reshape(_NW, nchunk, 128)
    mesh = plsc.VectorSubcoreMesh(core_axis_name="c", subcore_axis_name="s")

    @functools.partial(
        pl.kernel, mesh=mesh,
        out_type=jax.ShapeDtypeStruct((bn, d), jnp.float32),
        scratch_types=[
            pltpu.VMEM((nchunk, 128), jnp.int32),
            pltpu.VMEM((b_per_w, d), jnp.float32),
            pltpu.SemaphoreType.DMA,
        ],
    )
    def gather_k(table_hbm, idx_hbm, out_hbm, idx_v, rows_v, sem):
        wid = lax.axis_index("s") * _NC + lax.axis_index("c")
        base = wid * b_per_w
        pltpu.sync_copy(idx_hbm.at[wid], idx_v)
        cps = [pltpu.async_copy(table_hbm.at[idx_v.at[c]],
                                rows_v.at[pl.ds(c * 128, 128)], sem)
               for c in range(nchunk)]
        for cp in cps:
            cp.wait()
        pltpu.sync_copy(rows_v, out_hbm.at[pl.ds(base, b_per_w)])

    return gather_k(codebook_padded, idx_r)


def kernel(z, codebook):
    b, c, h, w = z.shape
    zp = jnp.transpose(z, (0, 2, 3, 1))
    z_flat = zp.reshape(-1, c)
    zbt = z_flat.astype(jnp.bfloat16).T
    zsq_row = jnp.sum(z_flat * z_flat, axis=1)[None, :]
    csq_wide = jnp.broadcast_to(
        jnp.sum(codebook * codebook, axis=1)[:, None],
        (codebook.shape[0], _BM))
    idx, loss_sum = _dist_argmin(codebook, zbt, zsq_row, csq_wide)
    cb_pad = jnp.pad(codebook, ((0, 0), (0, 128 - c)))
    q_flat = _sc_gather(cb_pad, idx)[:, :c]
    quantized = q_flat.reshape(zp.shape)
    mse = loss_sum[0, 0] / jnp.float32(z.size)
    total_loss = mse + 0.25 * mse
    quantized_st = zp + (quantized - zp)
    quantized_out = jnp.transpose(quantized_st, (0, 3, 1, 2))
    return (jnp.reshape(total_loss, ()), quantized_out, idx)


# speedup vs baseline: 1.2683x; 1.2683x over previous
import functools

import jax
import jax.numpy as jnp
from jax import lax
from jax.experimental import pallas as pl
from jax.experimental.pallas import tpu as pltpu
from jax.experimental.pallas import tpu_sc as plsc

_BM = 256
_BN = 2048

_NC = 2
_NS = 16
_NW = _NC * _NS


def _dist_argmin_body(cb_ref, zp_ref, zsq_ref,
                      idx_ref, loss_ref, runmin_ref, runidx_ref, truemin_ref):
    i = pl.program_id(0)
    j = pl.program_id(1)
    cb = cb_ref[...]
    mt = lax.dot_general(cb, zp_ref[...], (((1,), (0,)), ((), ())),
                         preferred_element_type=jnp.float32)
    csq = jnp.sum(cb * cb, axis=1, keepdims=True)
    d = (zsq_ref[...] + csq) - 2.0 * mt
    bmin = jnp.min(d, axis=0, keepdims=True)
    row = lax.broadcasted_iota(jnp.int32, d.shape, 0)
    bidx = jnp.min(jnp.where(d == bmin, row, jnp.int32(2**30)),
                   axis=0, keepdims=True) + j * _BN
    bmin_bf = bmin.astype(jnp.bfloat16).astype(jnp.float32)

    @pl.when(j == 0)
    def _init():
        runmin_ref[...] = bmin_bf
        runidx_ref[...] = bidx
        truemin_ref[...] = bmin

    @pl.when(j > 0)
    def _update():
        better = bmin < runmin_ref[...]
        runidx_ref[...] = jnp.where(better, bidx, runidx_ref[...])
        runmin_ref[...] = jnp.where(better, bmin_bf, runmin_ref[...])
        truemin_ref[...] = jnp.minimum(truemin_ref[...], bmin)

    @pl.when(j == pl.num_programs(1) - 1)
    def _emit():
        idx_ref[...] = runidx_ref[...].reshape(-1)
        s = jnp.sum(truemin_ref[...])

        @pl.when(i == 0)
        def _first():
            loss_ref[0, 0] = s

        @pl.when(i > 0)
        def _acc():
            loss_ref[0, 0] = loss_ref[0, 0] + s


def _dist_argmin(cb, zbt, zsq_row, interpret=False):
    nk, k2 = cb.shape
    n = zbt.shape[1]
    grid = (n // _BM, nk // _BN)
    return pl.pallas_call(
        _dist_argmin_body,
        grid=grid,
        in_specs=[
            pl.BlockSpec((_BN, k2), lambda i, j: (j, 0)),
            pl.BlockSpec((k2, _BM), lambda i, j: (0, i)),
            pl.BlockSpec((1, _BM), lambda i, j: (0, i)),
        ],
        out_specs=[
            pl.BlockSpec((_BM,), lambda i, j: (i,)),
            pl.BlockSpec(memory_space=pltpu.SMEM),
        ],
        out_shape=[
            jax.ShapeDtypeStruct((n,), jnp.int32),
            jax.ShapeDtypeStruct((1, 1), jnp.float32),
        ],
        scratch_shapes=[
            pltpu.VMEM((1, _BM), jnp.float32),
            pltpu.VMEM((1, _BM), jnp.int32),
            pltpu.VMEM((1, _BM), jnp.float32),
        ],
        compiler_params=pltpu.CompilerParams(
            dimension_semantics=("arbitrary", "arbitrary")),
        interpret=interpret,
    )(cb, zbt, zsq_row)


def _sc_gather(codebook_padded, idx):
    bn = idx.shape[0]
    d = codebook_padded.shape[1]
    b_per_w = bn // _NW
    nchunk = b_per_w // 128
    idx_r = idx.reshape(_NW, nchunk, 128)
    mesh = plsc.VectorSubcoreMesh(core_axis_name="c", subcore_axis_name="s")

    @functools.partial(
        pl.kernel, mesh=mesh,
        out_type=jax.ShapeDtypeStruct((bn, d), jnp.float32),
        scratch_types=[
            pltpu.VMEM((nchunk, 128), jnp.int32),
            pltpu.VMEM((b_per_w, d), jnp.float32),
            pltpu.SemaphoreType.DMA,
        ],
    )
    def gather_k(table_hbm, idx_hbm, out_hbm, idx_v, rows_v, sem):
        wid = lax.axis_index("s") * _NC + lax.axis_index("c")
        base = wid * b_per_w
        pltpu.sync_copy(idx_hbm.at[wid], idx_v)
        cps = [pltpu.async_copy(table_hbm.at[idx_v.at[c]],
                                rows_v.at[pl.ds(c * 128, 128)], sem)
               for c in range(nchunk)]
        for cp in cps:
            cp.wait()
        pltpu.sync_copy(rows_v, out_hbm.at[pl.ds(base, b_per_w)])

    return gather_k(codebook_padded, idx_r)


def kernel(z, codebook):
    b, c, h, w = z.shape
    zp = jnp.transpose(z, (0, 2, 3, 1))
    z_flat = zp.reshape(-1, c)
    zbt = z_flat.astype(jnp.bfloat16).T
    zsq_row = jnp.sum(z_flat * z_flat, axis=1)[None, :]
    idx, loss_sum = _dist_argmin(codebook, zbt, zsq_row)
    cb_pad = jnp.pad(codebook, ((0, 0), (0, 128 - c)))
    q_flat = _sc_gather(cb_pad, idx)[:, :c]
    quantized = q_flat.reshape(zp.shape)
    mse = loss_sum[0, 0] / jnp.float32(z.size)
    total_loss = mse + 0.25 * mse
    quantized_st = zp + (quantized - zp)
    quantized_out = jnp.transpose(quantized_st, (0, 3, 1, 2))
    return (jnp.reshape(total_loss, ()), quantized_out, idx)


# speedup vs baseline: 1.3587x; 1.0713x over previous
import functools

import jax
import jax.numpy as jnp
from jax import lax
from jax.experimental import pallas as pl
from jax.experimental.pallas import tpu as pltpu
from jax.experimental.pallas import tpu_sc as plsc

_BM = 256
_BN = 2048

_NC = 2
_NS = 16
_NW = _NC * _NS


_NSL = 4


def _dist_argmin_body(zb16_ref, z_ref, ct_ref,
                      idx_ref, loss_ref, runmin_ref, runidx_ref, truemin_ref):
    i = pl.program_id(0)
    j = pl.program_id(1)
    zb16 = zb16_ref[...]
    zb = z_ref[...]
    zsq = jnp.sum(zb * zb, axis=1, keepdims=True)
    bs = _BN // _NSL
    bmin = None
    bidx = None
    for s in range(_NSL):
        cb = ct_ref[:, pl.ds(s * bs, bs)]
        m = lax.dot_general(zb16, cb, (((1,), (0,)), ((), ())),
                            preferred_element_type=jnp.float32)
        csq = jnp.sum(cb * cb, axis=0, keepdims=True)
        d = (zsq + csq) - 2.0 * m
        smin = jnp.min(d, axis=1, keepdims=True)
        col = lax.broadcasted_iota(jnp.int32, d.shape, 1)
        sidx = jnp.min(jnp.where(d == smin, col, jnp.int32(2**30)),
                       axis=1, keepdims=True) + s * bs
        if s == 0:
            bmin, bidx = smin, sidx
        else:
            keep = bmin <= smin
            bidx = jnp.where(keep, bidx, sidx)
            bmin = jnp.where(keep, bmin, smin)
    bidx = bidx + j * _BN
    bmin_bf = bmin.astype(jnp.bfloat16).astype(jnp.float32)

    @pl.when(j == 0)
    def _init():
        runmin_ref[...] = bmin_bf
        runidx_ref[...] = bidx
        truemin_ref[...] = bmin

    @pl.when(j > 0)
    def _update():
        better = bmin < runmin_ref[...]
        runidx_ref[...] = jnp.where(better, bidx, runidx_ref[...])
        runmin_ref[...] = jnp.where(better, bmin_bf, runmin_ref[...])
        truemin_ref[...] = jnp.minimum(truemin_ref[...], bmin)

    @pl.when(j == pl.num_programs(1) - 1)
    def _emit():
        idx_ref[...] = runidx_ref[...].reshape(-1)
        s = jnp.sum(truemin_ref[...])

        @pl.when(i == 0)
        def _first():
            loss_ref[0, 0] = s

        @pl.when(i > 0)
        def _acc():
            loss_ref[0, 0] = loss_ref[0, 0] + s


def _dist_argmin(z_flat, ct, interpret=False):
    n, k = z_flat.shape
    nk = ct.shape[1]
    grid = (n // _BM, nk // _BN)
    zb16 = z_flat.astype(jnp.bfloat16)
    return pl.pallas_call(
        _dist_argmin_body,
        grid=grid,
        in_specs=[
            pl.BlockSpec((_BM, k), lambda i, j: (i, 0)),
            pl.BlockSpec((_BM, k), lambda i, j: (i, 0)),
            pl.BlockSpec((k, _BN), lambda i, j: (0, j)),
        ],
        out_specs=[
            pl.BlockSpec((_BM,), lambda i, j: (i,)),
            pl.BlockSpec(memory_space=pltpu.SMEM),
        ],
        out_shape=[
            jax.ShapeDtypeStruct((n,), jnp.int32),
            jax.ShapeDtypeStruct((1, 1), jnp.float32),
        ],
        scratch_shapes=[
            pltpu.VMEM((_BM, 1), jnp.float32),
            pltpu.VMEM((_BM, 1), jnp.int32),
            pltpu.VMEM((_BM, 1), jnp.float32),
        ],
        compiler_params=pltpu.CompilerParams(
            dimension_semantics=("arbitrary", "arbitrary")),
        interpret=interpret,
    )(zb16, z_flat, ct)


def _sc_gather(codebook_padded, idx):
    bn = idx.shape[0]
    d = codebook_padded.shape[1]
    b_per_w = bn // _NW
    nchunk = b_per_w // 128
    idx_r = idx.reshape(_NW, nchunk, 128)
    mesh = plsc.VectorSubcoreMesh(core_axis_name="c", subcore_axis_name="s")

    @functools.partial(
        pl.kernel, mesh=mesh,
        out_type=jax.ShapeDtypeStruct((bn, d), jnp.float32),
        scratch_types=[
            pltpu.VMEM((nchunk, 128), jnp.int32),
            pltpu.VMEM((b_per_w, d), jnp.float32),
            pltpu.SemaphoreType.DMA,
        ],
    )
    def gather_k(table_hbm, idx_hbm, out_hbm, idx_v, rows_v, sem):
        wid = lax.axis_index("s") * _NC + lax.axis_index("c")
        base = wid * b_per_w
        pltpu.sync_copy(idx_hbm.at[wid], idx_v)
        cps = [pltpu.async_copy(table_hbm.at[idx_v.at[c]],
                                rows_v.at[pl.ds(c * 128, 128)], sem)
               for c in range(nchunk)]
        for cp in cps:
            cp.wait()
        pltpu.sync_copy(rows_v, out_hbm.at[pl.ds(base, b_per_w)])

    return gather_k(codebook_padded, idx_r)


def kernel(z, codebook):
    b, c, h, w = z.shape
    zp = jnp.transpose(z, (0, 2, 3, 1))
    z_flat = zp.reshape(-1, c)
    ct = codebook.T
    idx, loss_sum = _dist_argmin(z_flat, ct)
    cb_pad = jnp.pad(codebook, ((0, 0), (0, 128 - c)))
    q_flat = _sc_gather(cb_pad, idx)[:, :c]
    quantized = q_flat.reshape(zp.shape)
    mse = loss_sum[0, 0] / jnp.float32(z.size)
    total_loss = mse + 0.25 * mse
    quantized_st = zp + (quantized - zp)
    quantized_out = jnp.transpose(quantized_st, (0, 3, 1, 2))
    return (jnp.reshape(total_loss, ()), quantized_out, idx)


# speedup vs baseline: 1.4250x; 1.0488x over previous
import functools

import jax
import jax.numpy as jnp
from jax import lax
from jax.experimental import pallas as pl
from jax.experimental.pallas import tpu as pltpu
from jax.experimental.pallas import tpu_sc as plsc

_BM = 512
_BN = 2048

_NC = 2
_NS = 16
_NW = _NC * _NS


_NSL = 4


def _dist_argmin_body(zb16_ref, z_ref, ct_ref,
                      idx_ref, loss_ref, runmin_ref, runidx_ref, truemin_ref):
    i = pl.program_id(0)
    j = pl.program_id(1)
    zb16 = zb16_ref[...]
    zb = z_ref[...]
    zsq = jnp.sum(zb * zb, axis=1, keepdims=True)
    bs = _BN // _NSL
    bmin = None
    bidx = None
    for s in range(_NSL):
        cb = ct_ref[:, pl.ds(s * bs, bs)]
        m = lax.dot_general(zb16, cb, (((1,), (0,)), ((), ())),
                            preferred_element_type=jnp.float32)
        csq = jnp.sum(cb * cb, axis=0, keepdims=True)
        d = (zsq + csq) - 2.0 * m
        smin = jnp.min(d, axis=1, keepdims=True)
        col = lax.broadcasted_iota(jnp.int32, d.shape, 1)
        sidx = jnp.min(jnp.where(d == smin, col, jnp.int32(2**30)),
                       axis=1, keepdims=True) + s * bs
        if s == 0:
            bmin, bidx = smin, sidx
        else:
            keep = bmin <= smin
            bidx = jnp.where(keep, bidx, sidx)
            bmin = jnp.where(keep, bmin, smin)
    bidx = bidx + j * _BN
    bmin_bf = bmin.astype(jnp.bfloat16).astype(jnp.float32)

    @pl.when(j == 0)
    def _init():
        runmin_ref[...] = bmin_bf
        runidx_ref[...] = bidx
        truemin_ref[...] = bmin

    @pl.when(j > 0)
    def _update():
        better = bmin < runmin_ref[...]
        runidx_ref[...] = jnp.where(better, bidx, runidx_ref[...])
        runmin_ref[...] = jnp.where(better, bmin_bf, runmin_ref[...])
        truemin_ref[...] = jnp.minimum(truemin_ref[...], bmin)

    @pl.when(j == pl.num_programs(1) - 1)
    def _emit():
        idx_ref[...] = runidx_ref[...].reshape(-1)
        s = jnp.sum(truemin_ref[...])

        @pl.when(i == 0)
        def _first():
            loss_ref[0, 0] = s

        @pl.when(i > 0)
        def _acc():
            loss_ref[0, 0] = loss_ref[0, 0] + s


def _dist_argmin(z_flat, ct, interpret=False):
    n, k = z_flat.shape
    nk = ct.shape[1]
    grid = (n // _BM, nk // _BN)
    zb16 = z_flat.astype(jnp.bfloat16)
    return pl.pallas_call(
        _dist_argmin_body,
        grid=grid,
        in_specs=[
            pl.BlockSpec((_BM, k), lambda i, j: (i, 0)),
            pl.BlockSpec((_BM, k), lambda i, j: (i, 0)),
            pl.BlockSpec((k, _BN), lambda i, j: (0, j)),
        ],
        out_specs=[
            pl.BlockSpec((_BM,), lambda i, j: (i,)),
            pl.BlockSpec(memory_space=pltpu.SMEM),
        ],
        out_shape=[
            jax.ShapeDtypeStruct((n,), jnp.int32),
            jax.ShapeDtypeStruct((1, 1), jnp.float32),
        ],
        scratch_shapes=[
            pltpu.VMEM((_BM, 1), jnp.float32),
            pltpu.VMEM((_BM, 1), jnp.int32),
            pltpu.VMEM((_BM, 1), jnp.float32),
        ],
        compiler_params=pltpu.CompilerParams(
            dimension_semantics=("arbitrary", "arbitrary")),
        interpret=interpret,
    )(zb16, z_flat, ct)


def _sc_gather(codebook_padded, idx):
    bn = idx.shape[0]
    d = codebook_padded.shape[1]
    b_per_w = bn // _NW
    nchunk = b_per_w // 128
    idx_r = idx.reshape(_NW, nchunk, 128)
    mesh = plsc.VectorSubcoreMesh(core_axis_name="c", subcore_axis_name="s")

    @functools.partial(
        pl.kernel, mesh=mesh,
        out_type=jax.ShapeDtypeStruct((bn, d), jnp.float32),
        scratch_types=[
            pltpu.VMEM((nchunk, 128), jnp.int32),
            pltpu.VMEM((b_per_w, d), jnp.float32),
            pltpu.SemaphoreType.DMA,
        ],
    )
    def gather_k(table_hbm, idx_hbm, out_hbm, idx_v, rows_v, sem):
        wid = lax.axis_index("s") * _NC + lax.axis_index("c")
        base = wid * b_per_w
        pltpu.sync_copy(idx_hbm.at[wid], idx_v)
        cps = [pltpu.async_copy(table_hbm.at[idx_v.at[c]],
                                rows_v.at[pl.ds(c * 128, 128)], sem)
               for c in range(nchunk)]
        for cp in cps:
            cp.wait()
        pltpu.sync_copy(rows_v, out_hbm.at[pl.ds(base, b_per_w)])

    return gather_k(codebook_padded, idx_r)


def kernel(z, codebook):
    b, c, h, w = z.shape
    zp = jnp.transpose(z, (0, 2, 3, 1))
    z_flat = zp.reshape(-1, c)
    ct = codebook.T
    idx, loss_sum = _dist_argmin(z_flat, ct)
    cb_pad = jnp.pad(codebook, ((0, 0), (0, 128 - c)))
    q_flat = _sc_gather(cb_pad, idx)[:, :c]
    quantized = q_flat.reshape(zp.shape)
    mse = loss_sum[0, 0] / jnp.float32(z.size)
    total_loss = mse + 0.25 * mse
    quantized_st = zp + (quantized - zp)
    quantized_out = jnp.transpose(quantized_st, (0, 3, 1, 2))
    return (jnp.reshape(total_loss, ()), quantized_out, idx)


# speedup vs baseline: 1.4908x; 1.0461x over previous
import functools

import jax
import jax.numpy as jnp
from jax import lax
from jax.experimental import pallas as pl
from jax.experimental.pallas import tpu as pltpu
from jax.experimental.pallas import tpu_sc as plsc

_BM = 1024
_BN = 2048

_NC = 2
_NS = 16
_NW = _NC * _NS


_NSL = 4


def _dist_argmin_body(zb16_ref, z_ref, ct_ref,
                      idx_ref, loss_ref, runmin_ref, runidx_ref, truemin_ref):
    i = pl.program_id(0)
    j = pl.program_id(1)
    zb16 = zb16_ref[...]
    zb = z_ref[...]
    zsq = jnp.sum(zb * zb, axis=1, keepdims=True)
    bs = _BN // _NSL
    bmin = None
    bidx = None
    for s in range(_NSL):
        cb = ct_ref[:, pl.ds(s * bs, bs)]
        m = lax.dot_general(zb16, cb, (((1,), (0,)), ((), ())),
                            preferred_element_type=jnp.float32)
        csq = jnp.sum(cb * cb, axis=0, keepdims=True)
        d = (zsq + csq) - 2.0 * m
        smin = jnp.min(d, axis=1, keepdims=True)
        col = lax.broadcasted_iota(jnp.int32, d.shape, 1)
        sidx = jnp.min(jnp.where(d == smin, col, jnp.int32(2**30)),
                       axis=1, keepdims=True) + s * bs
        if s == 0:
            bmin, bidx = smin, sidx
        else:
            keep = bmin <= smin
            bidx = jnp.where(keep, bidx, sidx)
            bmin = jnp.where(keep, bmin, smin)
    bidx = bidx + j * _BN
    bmin_bf = bmin.astype(jnp.bfloat16).astype(jnp.float32)

    @pl.when(j == 0)
    def _init():
        runmin_ref[...] = bmin_bf
        runidx_ref[...] = bidx
        truemin_ref[...] = bmin

    @pl.when(j > 0)
    def _update():
        better = bmin < runmin_ref[...]
        runidx_ref[...] = jnp.where(better, bidx, runidx_ref[...])
        runmin_ref[...] = jnp.where(better, bmin_bf, runmin_ref[...])
        truemin_ref[...] = jnp.minimum(truemin_ref[...], bmin)

    @pl.when(j == pl.num_programs(1) - 1)
    def _emit():
        idx_ref[...] = runidx_ref[...].reshape(-1)
        s = jnp.sum(truemin_ref[...])

        @pl.when(i == 0)
        def _first():
            loss_ref[0, 0] = s

        @pl.when(i > 0)
        def _acc():
            loss_ref[0, 0] = loss_ref[0, 0] + s


def _dist_argmin(z_flat, ct, interpret=False):
    n, k = z_flat.shape
    nk = ct.shape[1]
    grid = (n // _BM, nk // _BN)
    zb16 = z_flat.astype(jnp.bfloat16)
    return pl.pallas_call(
        _dist_argmin_body,
        grid=grid,
        in_specs=[
            pl.BlockSpec((_BM, k), lambda i, j: (i, 0)),
            pl.BlockSpec((_BM, k), lambda i, j: (i, 0)),
            pl.BlockSpec((k, _BN), lambda i, j: (0, j)),
        ],
        out_specs=[
            pl.BlockSpec((_BM,), lambda i, j: (i,)),
            pl.BlockSpec(memory_space=pltpu.SMEM),
        ],
        out_shape=[
            jax.ShapeDtypeStruct((n,), jnp.int32),
            jax.ShapeDtypeStruct((1, 1), jnp.float32),
        ],
        scratch_shapes=[
            pltpu.VMEM((_BM, 1), jnp.float32),
            pltpu.VMEM((_BM, 1), jnp.int32),
            pltpu.VMEM((_BM, 1), jnp.float32),
        ],
        compiler_params=pltpu.CompilerParams(
            dimension_semantics=("arbitrary", "arbitrary")),
        interpret=interpret,
    )(zb16, z_flat, ct)


def _sc_gather(codebook_padded, idx):
    bn = idx.shape[0]
    d = codebook_padded.shape[1]
    b_per_w = bn // _NW
    nchunk = b_per_w // 128
    idx_r = idx.reshape(_NW, nchunk, 128)
    mesh = plsc.VectorSubcoreMesh(core_axis_name="c", subcore_axis_name="s")

    @functools.partial(
        pl.kernel, mesh=mesh,
        out_type=jax.ShapeDtypeStruct((bn, d), jnp.float32),
        scratch_types=[
            pltpu.VMEM((nchunk, 128), jnp.int32),
            pltpu.VMEM((b_per_w, d), jnp.float32),
            pltpu.SemaphoreType.DMA,
        ],
    )
    def gather_k(table_hbm, idx_hbm, out_hbm, idx_v, rows_v, sem):
        wid = lax.axis_index("s") * _NC + lax.axis_index("c")
        base = wid * b_per_w
        pltpu.sync_copy(idx_hbm.at[wid], idx_v)
        cps = [pltpu.async_copy(table_hbm.at[idx_v.at[c]],
                                rows_v.at[pl.ds(c * 128, 128)], sem)
               for c in range(nchunk)]
        for cp in cps:
            cp.wait()
        pltpu.sync_copy(rows_v, out_hbm.at[pl.ds(base, b_per_w)])

    return gather_k(codebook_padded, idx_r)


def kernel(z, codebook):
    b, c, h, w = z.shape
    zp = jnp.transpose(z, (0, 2, 3, 1))
    z_flat = zp.reshape(-1, c)
    ct = codebook.T
    idx, loss_sum = _dist_argmin(z_flat, ct)
    cb_pad = jnp.pad(codebook, ((0, 0), (0, 128 - c)))
    q_flat = _sc_gather(cb_pad, idx)[:, :c]
    quantized = q_flat.reshape(zp.shape)
    mse = loss_sum[0, 0] / jnp.float32(z.size)
    total_loss = mse + 0.25 * mse
    quantized_st = zp + (quantized - zp)
    quantized_out = jnp.transpose(quantized_st, (0, 3, 1, 2))
    return (jnp.reshape(total_loss, ()), quantized_out, idx)


# speedup vs baseline: 1.5139x; 1.0155x over previous
import functools

import jax
import jax.numpy as jnp
from jax import lax
from jax.experimental import pallas as pl
from jax.experimental.pallas import tpu as pltpu
from jax.experimental.pallas import tpu_sc as plsc

_BM = 2048
_BN = 2048

_NC = 2
_NS = 16
_NW = _NC * _NS


_NSL = 4


def _dist_argmin_body(zb16_ref, z_ref, ct_ref,
                      idx_ref, loss_ref, runmin_ref, runidx_ref, truemin_ref):
    i = pl.program_id(0)
    j = pl.program_id(1)
    zb16 = zb16_ref[...]
    zb = z_ref[...]
    zsq = jnp.sum(zb * zb, axis=1, keepdims=True)
    bs = _BN // _NSL
    bmin = None
    bidx = None
    for s in range(_NSL):
        cb = ct_ref[:, pl.ds(s * bs, bs)]
        m = lax.dot_general(zb16, cb, (((1,), (0,)), ((), ())),
                            preferred_element_type=jnp.float32)
        csq = jnp.sum(cb * cb, axis=0, keepdims=True)
        d = (zsq + csq) - 2.0 * m
        smin = jnp.min(d, axis=1, keepdims=True)
        col = lax.broadcasted_iota(jnp.int32, d.shape, 1)
        sidx = jnp.min(jnp.where(d == smin, col, jnp.int32(2**30)),
                       axis=1, keepdims=True) + s * bs
        if s == 0:
            bmin, bidx = smin, sidx
        else:
            keep = bmin <= smin
            bidx = jnp.where(keep, bidx, sidx)
            bmin = jnp.where(keep, bmin, smin)
    bidx = bidx + j * _BN
    bmin_bf = bmin.astype(jnp.bfloat16).astype(jnp.float32)

    @pl.when(j == 0)
    def _init():
        runmin_ref[...] = bmin_bf
        runidx_ref[...] = bidx
        truemin_ref[...] = bmin

    @pl.when(j > 0)
    def _update():
        better = bmin < runmin_ref[...]
        runidx_ref[...] = jnp.where(better, bidx, runidx_ref[...])
        runmin_ref[...] = jnp.where(better, bmin_bf, runmin_ref[...])
        truemin_ref[...] = jnp.minimum(truemin_ref[...], bmin)

    @pl.when(j == pl.num_programs(1) - 1)
    def _emit():
        idx_ref[...] = runidx_ref[...].reshape(-1)
        s = jnp.sum(truemin_ref[...])

        @pl.when(i == 0)
        def _first():
            loss_ref[0, 0] = s

        @pl.when(i > 0)
        def _acc():
            loss_ref[0, 0] = loss_ref[0, 0] + s


def _dist_argmin(z_flat, ct, interpret=False):
    n, k = z_flat.shape
    nk = ct.shape[1]
    grid = (n // _BM, nk // _BN)
    zb16 = z_flat.astype(jnp.bfloat16)
    return pl.pallas_call(
        _dist_argmin_body,
        grid=grid,
        in_specs=[
            pl.BlockSpec((_BM, k), lambda i, j: (i, 0)),
            pl.BlockSpec((_BM, k), lambda i, j: (i, 0)),
            pl.BlockSpec((k, _BN), lambda i, j: (0, j)),
        ],
        out_specs=[
            pl.BlockSpec((_BM,), lambda i, j: (i,)),
            pl.BlockSpec(memory_space=pltpu.SMEM),
        ],
        out_shape=[
            jax.ShapeDtypeStruct((n,), jnp.int32),
            jax.ShapeDtypeStruct((1, 1), jnp.float32),
        ],
        scratch_shapes=[
            pltpu.VMEM((_BM, 1), jnp.float32),
            pltpu.VMEM((_BM, 1), jnp.int32),
            pltpu.VMEM((_BM, 1), jnp.float32),
        ],
        compiler_params=pltpu.CompilerParams(
            dimension_semantics=("arbitrary", "arbitrary")),
        interpret=interpret,
    )(zb16, z_flat, ct)


def _sc_gather(codebook_padded, idx):
    bn = idx.shape[0]
    d = codebook_padded.shape[1]
    b_per_w = bn // _NW
    nchunk = b_per_w // 128
    idx_r = idx.reshape(_NW, nchunk, 128)
    mesh = plsc.VectorSubcoreMesh(core_axis_name="c", subcore_axis_name="s")

    @functools.partial(
        pl.kernel, mesh=mesh,
        out_type=jax.ShapeDtypeStruct((bn, d), jnp.float32),
        scratch_types=[
            pltpu.VMEM((nchunk, 128), jnp.int32),
            pltpu.VMEM((b_per_w, d), jnp.float32),
            pltpu.SemaphoreType.DMA,
        ],
    )
    def gather_k(table_hbm, idx_hbm, out_hbm, idx_v, rows_v, sem):
        wid = lax.axis_index("s") * _NC + lax.axis_index("c")
        base = wid * b_per_w
        pltpu.sync_copy(idx_hbm.at[wid], idx_v)
        cps = [pltpu.async_copy(table_hbm.at[idx_v.at[c]],
                                rows_v.at[pl.ds(c * 128, 128)], sem)
               for c in range(nchunk)]
        for cp in cps:
            cp.wait()
        pltpu.sync_copy(rows_v, out_hbm.at[pl.ds(base, b_per_w)])

    return gather_k(codebook_padded, idx_r)


def kernel(z, codebook):
    b, c, h, w = z.shape
    zp = jnp.transpose(z, (0, 2, 3, 1))
    z_flat = zp.reshape(-1, c)
    ct = codebook.T
    idx, loss_sum = _dist_argmin(z_flat, ct)
    cb_pad = jnp.pad(codebook, ((0, 0), (0, 128 - c)))
    q_flat = _sc_gather(cb_pad, idx)[:, :c]
    quantized = q_flat.reshape(zp.shape)
    mse = loss_sum[0, 0] / jnp.float32(z.size)
    total_loss = mse + 0.25 * mse
    quantized_st = zp + (quantized - zp)
    quantized_out = jnp.transpose(quantized_st, (0, 3, 1, 2))
    return (jnp.reshape(total_loss, ()), quantized_out, idx)
